# baseline (speedup 1.0000x reference)
import jax
import jax.numpy as jnp
from jax import lax
from jax.experimental import pallas as pl
from jax.experimental.pallas import tpu as pltpu

_C1, _C2, _K = 8, 32, 5
_H_IN = 28
_HW = _H_IN * _H_IN
_H1 = _H_IN - _K + 1
_HP1 = _H1 // 2
_H2 = _HP1 - _K + 1
_HP2 = _H2 // 2
_NB = 256
_DT = jnp.bfloat16


def _pack_w1(w1):
    w1t = jnp.transpose(w1[:, 0, :, :], (0, 2, 1))
    ki = jnp.arange(32)[None, :]
    dr = jnp.arange(_H1)[:, None]
    ii = ki - dr
    valid = (ii >= 0) & (ii < _K)
    wb = w1t[:, :, jnp.clip(ii, 0, _K - 1)]
    wb = jnp.where(valid[None, None], wb, 0.0)
    wb = jnp.transpose(wb, (2, 0, 1, 3))
    return wb.reshape(_H1 * _C1, _K * 32).astype(_DT)


def _pack_w2(w2):
    w2t = jnp.transpose(w2, (0, 3, 2, 1))
    i = jnp.arange(_HP1)[None, :]
    dr = jnp.arange(_H2)[:, None]
    ii = i - dr
    valid = (ii >= 0) & (ii < _K)
    wb = w2t[:, :, jnp.clip(ii, 0, _K - 1), :]
    wb = jnp.where(valid[None, None, :, :, None], wb, 0.0)
    wb = jnp.transpose(wb, (2, 0, 1, 3, 4))
    return wb.reshape(_H2 * _C2, _K * _HP1 * _C1).astype(_DT)


def _net_kernel(x_ref, w1_ref, b1_ref, w2_ref, b2_ref, wf_ref, bf_ref, o_ref,
                xt_ref, p1_ref, p2_ref):
    f32 = jnp.float32
    nb = o_ref.shape[0]

    xv = x_ref[:, 0, :, :].astype(_DT).reshape(nb, _HW)
    t = lax.transpose(xv, (1, 0))
    for r in range(_H_IN):
        for c in range(_H_IN):
            p = r * _H_IN + c
            xt_ref[r:r + 1, c * nb:(c + 1) * nb] = t[p:p + 1, :]

    xw = xt_ref[...]
    ztail = jnp.zeros((4, _H1 * nb), _DT)
    for kj in range(_K):
        p1_ref[kj * 32:kj * 32 + _H_IN, :] = xw[:, kj * nb:(kj + _H1) * nb]
        p1_ref[kj * 32 + _H_IN:(kj + 1) * 32, :] = ztail
    c1 = jnp.dot(w1_ref[...], p1_ref[...], preferred_element_type=f32)
    c1 = jnp.maximum(c1 + b1_ref[...], 0.0)

    c1 = c1.reshape(_HP1, 2, _C1, _H1 * nb)
    r1 = jnp.maximum(c1[:, 0], c1[:, 1])
    pool1 = jnp.concatenate(
        [jnp.maximum(r1[:, :, (2 * j) * nb:(2 * j + 1) * nb],
                     r1[:, :, (2 * j + 1) * nb:(2 * j + 2) * nb])
         for j in range(_HP1)], axis=-1).astype(_DT)

    for kj in range(_K):
        p2_ref[kj * 96:(kj + 1) * 96, :] = (
            pool1[:, :, kj * nb:(kj + _H2) * nb].reshape(_HP1 * _C1, _H2 * nb))
    c2 = jnp.dot(w2_ref[...], p2_ref[...], preferred_element_type=f32)
    c2 = jnp.maximum(c2 + b2_ref[...], 0.0)

    c2 = c2.reshape(_HP2, 2, _C2, _H2 * nb)
    r2 = jnp.maximum(c2[:, 0], c2[:, 1])
    s = None
    for j2 in range(_HP2):
        tt = jnp.maximum(r2[:, :, (2 * j2) * nb:(2 * j2 + 1) * nb],
                         r2[:, :, (2 * j2 + 1) * nb:(2 * j2 + 2) * nb])
        s = tt if s is None else s + tt
    feat = (s[0] + s[1] + s[2] + s[3]) * (1.0 / 16.0)

    y = lax.dot_general(feat, wf_ref[...], (((0,), (1,)), ((), ())),
                        preferred_element_type=f32)
    o_ref[...] = (y + bf_ref[...]).astype(o_ref.dtype)


def kernel(x, w1, b1, w2, b2, wf, bf):
    B = x.shape[0]
    out_dim = wf.shape[0]
    nb = _NB
    n_blk = (B + nb - 1) // nb
    B_pad = n_blk * nb

    xk = x
    if B_pad != B:
        xk = jnp.pad(x, ((0, B_pad - B), (0, 0), (0, 0), (0, 0)))

    w1k = _pack_w1(w1)
    b1k = jnp.tile(b1, _H1).reshape(_H1 * _C1, 1)
    w2k = _pack_w2(w2)
    b2k = jnp.tile(b2, _H2).reshape(_H2 * _C2, 1)
    bfk = bf.reshape(1, out_dim)

    flops = B_pad * (2 * _C1 * _H1 * _H1 * _K * _K
                     + 2 * _C2 * _H2 * _H2 * _C1 * _K * _K
                     + 2 * out_dim * _C2)
    bytes_accessed = B_pad * (_HW + out_dim) * 4

    yt = pl.pallas_call(
        _net_kernel,
        out_shape=jax.ShapeDtypeStruct((B_pad, out_dim), jnp.float32),
        grid_spec=pltpu.PrefetchScalarGridSpec(
            num_scalar_prefetch=0,
            grid=(n_blk,),
            in_specs=[
                pl.BlockSpec((nb, 1, _H_IN, _H_IN), lambda g: (g, 0, 0, 0)),
                pl.BlockSpec((_H1 * _C1, _K * 32), lambda g: (0, 0)),
                pl.BlockSpec((_H1 * _C1, 1), lambda g: (0, 0)),
                pl.BlockSpec((_H2 * _C2, _K * _HP1 * _C1), lambda g: (0, 0)),
                pl.BlockSpec((_H2 * _C2, 1), lambda g: (0, 0)),
                pl.BlockSpec((out_dim, _C2), lambda g: (0, 0)),
                pl.BlockSpec((1, out_dim), lambda g: (0, 0)),
            ],
            out_specs=pl.BlockSpec((nb, out_dim), lambda g: (g, 0)),
            scratch_shapes=[
                pltpu.VMEM((_H_IN, _H_IN * nb), _DT),
                pltpu.VMEM((_K * 32, _H1 * nb), _DT),
                pltpu.VMEM((_K * _HP1 * _C1, _H2 * nb), _DT),
            ],
        ),
        compiler_params=pltpu.CompilerParams(
            dimension_semantics=("arbitrary",),
            vmem_limit_bytes=48 * 1024 * 1024),
        cost_estimate=pl.CostEstimate(flops=flops, transcendentals=0,
                                      bytes_accessed=bytes_accessed),
    )(xk, w1k, b1k, w2k, b2k, wf, bfk)
    return yt[:B]


# speedup vs baseline: 1.6879x; 1.6879x over previous
import jax
import jax.numpy as jnp
from jax import lax
from jax.experimental import pallas as pl
from jax.experimental.pallas import tpu as pltpu

_C1, _C2, _K = 8, 32, 5
_H_IN = 28
_HW = _H_IN * _H_IN
_H1 = _H_IN - _K + 1
_HP1 = _H1 // 2
_H2 = _HP1 - _K + 1
_HP2 = _H2 // 2
_NB = 256
_DT = jnp.bfloat16


def _pack_w1(w1):
    w1t = jnp.transpose(w1[:, 0, :, :], (0, 2, 1))
    ki = jnp.arange(32)[None, :]
    dr = jnp.arange(_H1)[:, None]
    ii = ki - dr
    valid = (ii >= 0) & (ii < _K)
    wb = w1t[:, :, jnp.clip(ii, 0, _K - 1)]
    wb = jnp.where(valid[None, None], wb, 0.0)
    wb = jnp.transpose(wb, (2, 0, 1, 3))
    return wb.reshape(_H1 * _C1, _K * 32).astype(_DT)


def _pack_w2(w2):
    w2t = jnp.transpose(w2, (0, 3, 2, 1))
    i = jnp.arange(_HP1)[None, :]
    dr = jnp.arange(_H2)[:, None]
    ii = i - dr
    valid = (ii >= 0) & (ii < _K)
    wb = w2t[:, :, jnp.clip(ii, 0, _K - 1), :]
    wb = jnp.where(valid[None, None, :, :, None], wb, 0.0)
    wb = jnp.transpose(wb, (2, 0, 1, 3, 4))
    return wb.reshape(_H2 * _C2, _K * _HP1 * _C1).astype(_DT)


def _net_kernel(x_ref, w1_ref, b1_ref, w2_ref, b2_ref, wf_ref, bf_ref, o_ref,
                xt_ref, p1_ref, p2_ref):
    f32 = jnp.float32
    nb = o_ref.shape[0]

    xv = x_ref[0].astype(_DT).reshape(nb, _HW)
    t = lax.transpose(xv, (1, 0))
    for r in range(_H_IN):
        for c in range(_H_IN):
            p = r * _H_IN + c
            xt_ref[r:r + 1, c * nb:(c + 1) * nb] = t[p:p + 1, :]

    xw = xt_ref[...]
    ztail = jnp.zeros((4, _H1 * nb), _DT)
    for kj in range(_K):
        p1_ref[kj * 32:kj * 32 + _H_IN, :] = xw[:, kj * nb:(kj + _H1) * nb]
        p1_ref[kj * 32 + _H_IN:(kj + 1) * 32, :] = ztail
    c1 = jnp.dot(w1_ref[...], p1_ref[...], preferred_element_type=f32)
    c1 = jnp.maximum(c1 + b1_ref[...], 0.0)

    c1 = c1.reshape(_HP1, 2, _C1, _H1 * nb)
    r1 = jnp.maximum(c1[:, 0], c1[:, 1])
    pool1 = jnp.concatenate(
        [jnp.maximum(r1[:, :, (2 * j) * nb:(2 * j + 1) * nb],
                     r1[:, :, (2 * j + 1) * nb:(2 * j + 2) * nb])
         for j in range(_HP1)], axis=-1).astype(_DT)

    for kj in range(_K):
        p2_ref[kj * 96:(kj + 1) * 96, :] = (
            pool1[:, :, kj * nb:(kj + _H2) * nb].reshape(_HP1 * _C1, _H2 * nb))
    c2 = jnp.dot(w2_ref[...], p2_ref[...], preferred_element_type=f32)
    c2 = jnp.maximum(c2 + b2_ref[...], 0.0)

    c2 = c2.reshape(_HP2, 2, _C2, _H2 * nb)
    r2 = jnp.maximum(c2[:, 0], c2[:, 1])
    s = None
    for j2 in range(_HP2):
        tt = jnp.maximum(r2[:, :, (2 * j2) * nb:(2 * j2 + 1) * nb],
                         r2[:, :, (2 * j2 + 1) * nb:(2 * j2 + 2) * nb])
        s = tt if s is None else s + tt
    feat = (s[0] + s[1] + s[2] + s[3]) * (1.0 / 16.0)

    y = lax.dot_general(feat, wf_ref[...], (((0,), (1,)), ((), ())),
                        preferred_element_type=f32)
    o_ref[...] = (y + bf_ref[...]).astype(o_ref.dtype)


def kernel(x, w1, b1, w2, b2, wf, bf):
    B = x.shape[0]
    out_dim = wf.shape[0]
    nb = _NB
    n_blk = (B + nb - 1) // nb
    B_pad = n_blk * nb

    xs = x[:, 0]
    if B_pad != B:
        xs = jnp.pad(xs, ((0, B_pad - B), (0, 0), (0, 0)))
    xk = xs.reshape(n_blk, nb, _H_IN, _H_IN)

    w1k = _pack_w1(w1)
    b1k = jnp.tile(b1, _H1).reshape(_H1 * _C1, 1)
    w2k = _pack_w2(w2)
    b2k = jnp.tile(b2, _H2).reshape(_H2 * _C2, 1)
    bfk = bf.reshape(1, out_dim)

    flops = B_pad * (2 * _C1 * _H1 * _H1 * _K * _K
                     + 2 * _C2 * _H2 * _H2 * _C1 * _K * _K
                     + 2 * out_dim * _C2)
    bytes_accessed = B_pad * (_HW + out_dim) * 4

    yt = pl.pallas_call(
        _net_kernel,
        out_shape=jax.ShapeDtypeStruct((B_pad, out_dim), jnp.float32),
        grid_spec=pltpu.PrefetchScalarGridSpec(
            num_scalar_prefetch=0,
            grid=(n_blk,),
            in_specs=[
                pl.BlockSpec((1, nb, _H_IN, _H_IN), lambda g: (g, 0, 0, 0)),
                pl.BlockSpec((_H1 * _C1, _K * 32), lambda g: (0, 0)),
                pl.BlockSpec((_H1 * _C1, 1), lambda g: (0, 0)),
                pl.BlockSpec((_H2 * _C2, _K * _HP1 * _C1), lambda g: (0, 0)),
                pl.BlockSpec((_H2 * _C2, 1), lambda g: (0, 0)),
                pl.BlockSpec((out_dim, _C2), lambda g: (0, 0)),
                pl.BlockSpec((1, out_dim), lambda g: (0, 0)),
            ],
            out_specs=pl.BlockSpec((nb, out_dim), lambda g: (g, 0)),
            scratch_shapes=[
                pltpu.VMEM((_H_IN, _H_IN * nb), _DT),
                pltpu.VMEM((_K * 32, _H1 * nb), _DT),
                pltpu.VMEM((_K * _HP1 * _C1, _H2 * nb), _DT),
            ],
        ),
        compiler_params=pltpu.CompilerParams(
            dimension_semantics=("arbitrary",),
            vmem_limit_bytes=48 * 1024 * 1024),
        cost_estimate=pl.CostEstimate(flops=flops, transcendentals=0,
                                      bytes_accessed=bytes_accessed),
    )(xk, w1k, b1k, w2k, b2k, wf, bfk)
    return yt[:B]


# speedup vs baseline: 1.7498x; 1.0367x over previous
import jax
import jax.numpy as jnp
from jax import lax
from jax.experimental import pallas as pl
from jax.experimental.pallas import tpu as pltpu

_C1, _C2, _K = 8, 32, 5
_H_IN = 28
_HW = _H_IN * _H_IN
_H1 = _H_IN - _K + 1
_HP1 = _H1 // 2
_H2 = _HP1 - _K + 1
_HP2 = _H2 // 2
_NB = 512
_DT = jnp.bfloat16


def _pack_w1(w1):
    w1t = jnp.transpose(w1[:, 0, :, :], (0, 2, 1))
    ki = jnp.arange(32)[None, :]
    dr = jnp.arange(_H1)[:, None]
    ii = ki - dr
    valid = (ii >= 0) & (ii < _K)
    wb = w1t[:, :, jnp.clip(ii, 0, _K - 1)]
    wb = jnp.where(valid[None, None], wb, 0.0)
    wb = jnp.transpose(wb, (2, 0, 1, 3))
    return wb.reshape(_H1 * _C1, _K * 32).astype(_DT)


def _pack_w2(w2):
    w2t = jnp.transpose(w2, (0, 3, 2, 1))
    i = jnp.arange(_HP1)[None, :]
    dr = jnp.arange(_H2)[:, None]
    ii = i - dr
    valid = (ii >= 0) & (ii < _K)
    wb = w2t[:, :, jnp.clip(ii, 0, _K - 1), :]
    wb = jnp.where(valid[None, None, :, :, None], wb, 0.0)
    wb = jnp.transpose(wb, (2, 0, 1, 3, 4))
    return wb.reshape(_H2 * _C2, _K * _HP1 * _C1).astype(_DT)


def _net_kernel(x_ref, w1_ref, b1_ref, w2_ref, b2_ref, wf_ref, bf_ref, o_ref,
                xt_ref, p1_ref, p2_ref):
    f32 = jnp.float32
    nb = o_ref.shape[0]

    xv = x_ref[0].astype(_DT).reshape(nb, _HW)
    t = lax.transpose(xv, (1, 0))
    for r in range(_H_IN):
        for c in range(_H_IN):
            p = r * _H_IN + c
            xt_ref[r:r + 1, c * nb:(c + 1) * nb] = t[p:p + 1, :]

    xw = xt_ref[...]
    ztail = jnp.zeros((4, _H1 * nb), _DT)
    for kj in range(_K):
        p1_ref[kj * 32:kj * 32 + _H_IN, :] = xw[:, kj * nb:(kj + _H1) * nb]
        p1_ref[kj * 32 + _H_IN:(kj + 1) * 32, :] = ztail
    c1 = jnp.dot(w1_ref[...], p1_ref[...], preferred_element_type=f32)
    c1 = jnp.maximum(c1 + b1_ref[...], 0.0)

    c1 = c1.reshape(_HP1, 2, _C1, _H1 * nb)
    r1 = jnp.maximum(c1[:, 0], c1[:, 1])
    pool1 = jnp.concatenate(
        [jnp.maximum(r1[:, :, (2 * j) * nb:(2 * j + 1) * nb],
                     r1[:, :, (2 * j + 1) * nb:(2 * j + 2) * nb])
         for j in range(_HP1)], axis=-1).astype(_DT)

    for kj in range(_K):
        p2_ref[kj * 96:(kj + 1) * 96, :] = (
            pool1[:, :, kj * nb:(kj + _H2) * nb].reshape(_HP1 * _C1, _H2 * nb))
    c2 = jnp.dot(w2_ref[...], p2_ref[...], preferred_element_type=f32)
    c2 = jnp.maximum(c2 + b2_ref[...], 0.0)

    c2 = c2.reshape(_HP2, 2, _C2, _H2 * nb)
    r2 = jnp.maximum(c2[:, 0], c2[:, 1])
    s = None
    for j2 in range(_HP2):
        tt = jnp.maximum(r2[:, :, (2 * j2) * nb:(2 * j2 + 1) * nb],
                         r2[:, :, (2 * j2 + 1) * nb:(2 * j2 + 2) * nb])
        s = tt if s is None else s + tt
    feat = (s[0] + s[1] + s[2] + s[3]) * (1.0 / 16.0)

    y = lax.dot_general(feat, wf_ref[...], (((0,), (1,)), ((), ())),
                        preferred_element_type=f32)
    o_ref[...] = (y + bf_ref[...]).astype(o_ref.dtype)


def kernel(x, w1, b1, w2, b2, wf, bf):
    B = x.shape[0]
    out_dim = wf.shape[0]
    nb = _NB
    n_blk = (B + nb - 1) // nb
    B_pad = n_blk * nb

    xs = x[:, 0]
    if B_pad != B:
        xs = jnp.pad(xs, ((0, B_pad - B), (0, 0), (0, 0)))
    xk = xs.reshape(n_blk, nb, _H_IN, _H_IN)

    w1k = _pack_w1(w1)
    b1k = jnp.tile(b1, _H1).reshape(_H1 * _C1, 1)
    w2k = _pack_w2(w2)
    b2k = jnp.tile(b2, _H2).reshape(_H2 * _C2, 1)
    bfk = bf.reshape(1, out_dim)

    flops = B_pad * (2 * _C1 * _H1 * _H1 * _K * _K
                     + 2 * _C2 * _H2 * _H2 * _C1 * _K * _K
                     + 2 * out_dim * _C2)
    bytes_accessed = B_pad * (_HW + out_dim) * 4

    yt = pl.pallas_call(
        _net_kernel,
        out_shape=jax.ShapeDtypeStruct((B_pad, out_dim), jnp.float32),
        grid_spec=pltpu.PrefetchScalarGridSpec(
            num_scalar_prefetch=0,
            grid=(n_blk,),
            in_specs=[
                pl.BlockSpec((1, nb, _H_IN, _H_IN), lambda g: (g, 0, 0, 0)),
                pl.BlockSpec((_H1 * _C1, _K * 32), lambda g: (0, 0)),
                pl.BlockSpec((_H1 * _C1, 1), lambda g: (0, 0)),
                pl.BlockSpec((_H2 * _C2, _K * _HP1 * _C1), lambda g: (0, 0)),
                pl.BlockSpec((_H2 * _C2, 1), lambda g: (0, 0)),
                pl.BlockSpec((out_dim, _C2), lambda g: (0, 0)),
                pl.BlockSpec((1, out_dim), lambda g: (0, 0)),
            ],
            out_specs=pl.BlockSpec((nb, out_dim), lambda g: (g, 0)),
            scratch_shapes=[
                pltpu.VMEM((_H_IN, _H_IN * nb), _DT),
                pltpu.VMEM((_K * 32, _H1 * nb), _DT),
                pltpu.VMEM((_K * _HP1 * _C1, _H2 * nb), _DT),
            ],
        ),
        compiler_params=pltpu.CompilerParams(
            dimension_semantics=("arbitrary",),
            vmem_limit_bytes=48 * 1024 * 1024),
        cost_estimate=pl.CostEstimate(flops=flops, transcendentals=0,
                                      bytes_accessed=bytes_accessed),
    )(xk, w1k, b1k, w2k, b2k, wf, bfk)
    return yt[:B]
